# initial kernel scaffold (unmeasured)
import jax
import jax.numpy as jnp
from jax import lax
from jax.experimental import pallas as pl
from jax.experimental.pallas import tpu as pltpu

TOKENS = 4096
DIM = 2048
OUT_TILE = 512


def kernel(ids, E):
    vshard = E.shape[0]
    my_x = lax.axis_index("x")
    base = my_x * vshard

    rows = jnp.clip(ids - base, 0, vshard - 1)
    mask = (ids >= base) & (ids < base + vshard)
    partial = (E[rows] * mask[:, None]).astype(jnp.bfloat16)

    def body(partial_ref, out_ref, recv_ref, stage_ref, send_sem, recv_sem, out_sem):
        x = lax.axis_index("x")
        y = lax.axis_index("y")
        z = lax.axis_index("z")

        rdma = pltpu.make_async_remote_copy(
            src_ref=partial_ref,
            dst_ref=recv_ref,
            send_sem=send_sem,
            recv_sem=recv_sem,
            device_id=(1 - x, y, z),
            device_id_type=pl.DeviceIdType.MESH,
        )
        rdma.start()
        rdma.wait()

        for k in range(TOKENS // OUT_TILE):
            sl = pl.ds(k * OUT_TILE, OUT_TILE)
            stage_ref[...] = partial_ref[sl, :].astype(jnp.float32) + recv_ref[
                sl, :
            ].astype(jnp.float32)
            copy = pltpu.make_async_copy(stage_ref, out_ref.at[sl, :], out_sem)
            copy.start()
            copy.wait()

    return pl.pallas_call(
        body,
        out_shape=jax.ShapeDtypeStruct((TOKENS, DIM), jnp.float32),
        in_specs=[pl.BlockSpec(memory_space=pltpu.VMEM)],
        out_specs=pl.BlockSpec(memory_space=pltpu.ANY),
        scratch_shapes=[
            pltpu.VMEM((TOKENS, DIM), jnp.bfloat16),
            pltpu.VMEM((OUT_TILE, DIM), jnp.float32),
            pltpu.SemaphoreType.DMA,
            pltpu.SemaphoreType.DMA,
            pltpu.SemaphoreType.DMA,
        ],
        compiler_params=pltpu.CompilerParams(collective_id=0),
    )(partial)


# baseline (device time: 2732046 ns/iter reference)
import jax
import jax.numpy as jnp
from jax import lax
from jax.experimental import pallas as pl
from jax.experimental.pallas import tpu as pltpu

TOKENS = 4096
DIM = 2048
OUT_TILE = 512


def kernel(ids, E):
    vshard = E.shape[0]
    my_x = lax.axis_index("x")
    base = my_x * vshard

    rows = jnp.clip(ids - base, 0, vshard - 1)
    mask = (ids >= base) & (ids < base + vshard)
    partial = (E[rows] * mask[:, None]).astype(jnp.bfloat16)

    def body(partial_ref, out_ref, recv_ref, stage_ref, send_sem, recv_sem, out_sem):
        x = lax.axis_index("x")
        y = lax.axis_index("y")
        z = lax.axis_index("z")

        rdma = pltpu.make_async_remote_copy(
            src_ref=partial_ref,
            dst_ref=recv_ref,
            send_sem=send_sem,
            recv_sem=recv_sem,
            device_id=(1 - x, y, z),
            device_id_type=pl.DeviceIdType.MESH,
        )
        rdma.start()
        rdma.wait()

        for k in range(TOKENS // OUT_TILE):
            sl = pl.ds(k * OUT_TILE, OUT_TILE)
            stage_ref[...] = partial_ref[sl, :].astype(jnp.float32) + recv_ref[
                sl, :
            ].astype(jnp.float32)
            copy = pltpu.make_async_copy(stage_ref, out_ref.at[sl, :], out_sem)
            copy.start()
            copy.wait()

    return pl.pallas_call(
        body,
        out_shape=jax.ShapeDtypeStruct((TOKENS, DIM), jnp.float32),
        in_specs=[pl.BlockSpec(memory_space=pltpu.VMEM)],
        out_specs=pl.BlockSpec(memory_space=pl.ANY),
        scratch_shapes=[
            pltpu.VMEM((TOKENS, DIM), jnp.bfloat16),
            pltpu.VMEM((OUT_TILE, DIM), jnp.float32),
            pltpu.SemaphoreType.DMA,
            pltpu.SemaphoreType.DMA,
            pltpu.SemaphoreType.DMA,
        ],
    )(partial)


# device time: 255414 ns/iter; 10.6965x vs baseline; 10.6965x over previous
import jax
import jax.numpy as jnp
from jax import lax
from jax.experimental import pallas as pl
from jax.experimental.pallas import tpu as pltpu

TOKENS = 4096
DIM = 2048
NY = 4
NZ = 4
CHUNK = TOKENS // (NY * NZ)
BLOCK = NZ * CHUNK
OUT_TILE = 512


def kernel(ids, E):
    vshard = E.shape[0]
    my_x = lax.axis_index("x")
    my_y = lax.axis_index("y")
    my_z = lax.axis_index("z")
    base = my_x * vshard

    rows_all = jnp.clip(ids - base, 0, vshard - 1).astype(jnp.int32)
    mask_all = ((ids >= base) & (ids < base + vshard)).astype(jnp.float32)
    c = my_y * NZ + my_z
    rows = lax.dynamic_slice(rows_all, (c * CHUNK,), (CHUNK,))
    mask = lax.dynamic_slice(mask_all, (c * CHUNK,), (CHUNK,))[:, None]

    def body(
        rows_ref,
        mask_ref,
        e_ref,
        out_ref,
        ag_ref,
        stage_ref,
        xrecv_ref,
        ostage_ref,
        gsem,
        xs_sem,
        xr_sem,
        zs_sems,
        zr_sems,
        ys_sems,
        yr_sems,
        osem,
    ):
        x = lax.axis_index("x")
        y = lax.axis_index("y")
        z = lax.axis_index("z")
        my_off = (y * NZ + z) * CHUNK

        def issue(t, _):
            r = rows_ref[t]
            pltpu.make_async_copy(
                e_ref.at[pl.ds(r, 1), :], stage_ref.at[pl.ds(t, 1), :], gsem
            ).start()
            return 0

        lax.fori_loop(0, CHUNK, issue, 0)

        def drain(t, _):
            pltpu.make_async_copy(
                e_ref.at[pl.ds(0, 1), :], stage_ref.at[pl.ds(t, 1), :], gsem
            ).wait()
            return 0

        lax.fori_loop(0, CHUNK, drain, 0)

        ag_ref[pl.ds(my_off, CHUNK), :] = (
            stage_ref[...] * mask_ref[...]
        ).astype(jnp.bfloat16)

        x_rdma = pltpu.make_async_remote_copy(
            src_ref=ag_ref.at[pl.ds(my_off, CHUNK), :],
            dst_ref=xrecv_ref,
            send_sem=xs_sem,
            recv_sem=xr_sem,
            device_id=(1 - x, y, z),
            device_id_type=pl.DeviceIdType.MESH,
        )
        x_rdma.start()
        x_rdma.wait()
        ag_ref[pl.ds(my_off, CHUNK), :] = (
            ag_ref[pl.ds(my_off, CHUNK), :] + xrecv_ref[...]
        )

        for h in range(NZ - 1):
            src_z = jnp.mod(z - h + NZ, NZ)
            off = (y * NZ + src_z) * CHUNK
            rdma = pltpu.make_async_remote_copy(
                src_ref=ag_ref.at[pl.ds(off, CHUNK), :],
                dst_ref=ag_ref.at[pl.ds(off, CHUNK), :],
                send_sem=zs_sems.at[h],
                recv_sem=zr_sems.at[h],
                device_id=(x, y, jnp.mod(z + 1, NZ)),
                device_id_type=pl.DeviceIdType.MESH,
            )
            rdma.start()
            rdma.wait()

        for h in range(NY - 1):
            src_y = jnp.mod(y - h + NY, NY)
            off = src_y * BLOCK
            rdma = pltpu.make_async_remote_copy(
                src_ref=ag_ref.at[pl.ds(off, BLOCK), :],
                dst_ref=ag_ref.at[pl.ds(off, BLOCK), :],
                send_sem=ys_sems.at[h],
                recv_sem=yr_sems.at[h],
                device_id=(x, jnp.mod(y + 1, NY), z),
                device_id_type=pl.DeviceIdType.MESH,
            )
            rdma.start()
            rdma.wait()

        for k in range(TOKENS // OUT_TILE):
            sl = pl.ds(k * OUT_TILE, OUT_TILE)
            ostage_ref[...] = ag_ref[sl, :].astype(jnp.float32)
            copy = pltpu.make_async_copy(ostage_ref, out_ref.at[sl, :], osem)
            copy.start()
            copy.wait()

    return pl.pallas_call(
        body,
        out_shape=jax.ShapeDtypeStruct((TOKENS, DIM), jnp.float32),
        in_specs=[
            pl.BlockSpec(memory_space=pltpu.SMEM),
            pl.BlockSpec(memory_space=pltpu.VMEM),
            pl.BlockSpec(memory_space=pl.ANY),
        ],
        out_specs=pl.BlockSpec(memory_space=pl.ANY),
        scratch_shapes=[
            pltpu.VMEM((TOKENS, DIM), jnp.bfloat16),
            pltpu.VMEM((CHUNK, DIM), jnp.float32),
            pltpu.VMEM((CHUNK, DIM), jnp.bfloat16),
            pltpu.VMEM((OUT_TILE, DIM), jnp.float32),
            pltpu.SemaphoreType.DMA,
            pltpu.SemaphoreType.DMA,
            pltpu.SemaphoreType.DMA,
            pltpu.SemaphoreType.DMA((NZ - 1,)),
            pltpu.SemaphoreType.DMA((NZ - 1,)),
            pltpu.SemaphoreType.DMA((NY - 1,)),
            pltpu.SemaphoreType.DMA((NY - 1,)),
            pltpu.SemaphoreType.DMA,
        ],
    )(rows, mask, E)


# device time: 167863 ns/iter; 16.2755x vs baseline; 1.5216x over previous
import jax
import jax.numpy as jnp
from jax import lax
from jax.experimental import pallas as pl
from jax.experimental.pallas import tpu as pltpu

TOKENS = 4096
DIM = 2048
HALF = DIM // 2
NY = 4
NZ = 4
CHUNK = TOKENS // (NY * NZ)
BLOCK = NZ * CHUNK
N_SWEEP = 4
N_HOP = 3


def kernel(ids, E):
    vshard = E.shape[0]
    my_x = lax.axis_index("x")
    my_y = lax.axis_index("y")
    my_z = lax.axis_index("z")
    base = my_x * vshard

    rows_all = jnp.clip(ids - base, 0, vshard - 1).astype(jnp.int32)
    mask_all = ((ids >= base) & (ids < base + vshard)).astype(jnp.float32)
    c = my_y * NZ + my_z
    rows = lax.dynamic_slice(rows_all, (c * CHUNK,), (CHUNK,))
    mask = lax.dynamic_slice(mask_all, (c * CHUNK,), (CHUNK,))[:, None]

    def body(
        rows_ref,
        mask_ref,
        e_ref,
        out_ref,
        ag_ref,
        stage_ref,
        xrecv_ref,
        ostage_ref,
        gsem,
        xs_sem,
        xr_sem,
        ps_sems,
        pr_sems,
        ms_sems,
        mr_sems,
        osems,
    ):
        x = lax.axis_index("x")
        y = lax.axis_index("y")
        z = lax.axis_index("z")

        def issue(t, _):
            r = rows_ref[t]
            pltpu.make_async_copy(
                e_ref.at[pl.ds(r, 1), :], stage_ref.at[pl.ds(t, 1), :], gsem
            ).start()
            return 0

        lax.fori_loop(0, CHUNK, issue, 0)

        def drain(t, _):
            pltpu.make_async_copy(
                e_ref.at[pl.ds(0, 1), :], stage_ref.at[pl.ds(t, 1), :], gsem
            ).wait()
            return 0

        lax.fori_loop(0, CHUNK, drain, 0)

        home = ag_ref.at[pl.ds(y, 1), pl.ds(z, 1), :, :]
        ag_ref[pl.ds(y, 1), pl.ds(z, 1), :, :] = jnp.reshape(
            (stage_ref[...] * mask_ref[...]).astype(jnp.bfloat16),
            (1, 1, CHUNK, DIM),
        )

        x_rdma = pltpu.make_async_remote_copy(
            src_ref=home,
            dst_ref=xrecv_ref,
            send_sem=xs_sem,
            recv_sem=xr_sem,
            device_id=(1 - x, y, z),
            device_id_type=pl.DeviceIdType.MESH,
        )
        x_rdma.start()
        x_rdma.wait()
        ag_ref[pl.ds(y, 1), pl.ds(z, 1), :, :] = (
            ag_ref[pl.ds(y, 1), pl.ds(z, 1), :, :] + xrecv_ref[...]
        )

        def sweep_slice(s, b):
            bb = jnp.clip(b, 0, 3)
            if s == 0:
                return ag_ref.at[pl.ds(y, 1), pl.ds(bb, 1), :, pl.ds(0, HALF)]
            if s == 1:
                return ag_ref.at[pl.ds(bb, 1), pl.ds(z, 1), :, pl.ds(HALF, HALF)]
            if s == 2:
                return ag_ref.at[pl.ds(bb, 1), :, :, pl.ds(0, HALF)]
            return ag_ref.at[:, pl.ds(bb, 1), :, pl.ds(HALF, HALF)]

        def sweep_pos(s):
            return z if s in (0, 3) else y

        def sweep_dev(s, delta):
            if s in (0, 3):
                return (x, y, jnp.clip(z + delta, 0, NZ - 1))
            return (x, jnp.clip(y + delta, 0, NY - 1), z)

        deferred_sends = []

        def run_phase(sweeps):
            for h in range(N_HOP):
                for s in sweeps:
                    p = sweep_pos(s)
                    cond_p = (p < 3) & (p - h >= 0)
                    rp = pltpu.make_async_remote_copy(
                        src_ref=sweep_slice(s, p - h),
                        dst_ref=sweep_slice(s, p - h),
                        send_sem=ps_sems.at[s, h],
                        recv_sem=pr_sems.at[s, h],
                        device_id=sweep_dev(s, 1),
                        device_id_type=pl.DeviceIdType.MESH,
                    )

                    @pl.when(cond_p)
                    def _(rp=rp):
                        rp.start()

                    deferred_sends.append((cond_p, rp))

                    cond_m = (p > 0) & (p + h <= 3)
                    rm = pltpu.make_async_remote_copy(
                        src_ref=sweep_slice(s, p + h),
                        dst_ref=sweep_slice(s, p + h),
                        send_sem=ms_sems.at[s, h],
                        recv_sem=mr_sems.at[s, h],
                        device_id=sweep_dev(s, -1),
                        device_id_type=pl.DeviceIdType.MESH,
                    )

                    @pl.when(cond_m)
                    def _(rm=rm):
                        rm.start()

                    deferred_sends.append((cond_m, rm))

                for s in sweeps:
                    p = sweep_pos(s)
                    cond_r = p - h - 1 >= 0
                    wr = pltpu.make_async_remote_copy(
                        src_ref=sweep_slice(s, p - h - 1),
                        dst_ref=sweep_slice(s, p - h - 1),
                        send_sem=ps_sems.at[s, h],
                        recv_sem=pr_sems.at[s, h],
                        device_id=sweep_dev(s, 1),
                        device_id_type=pl.DeviceIdType.MESH,
                    )

                    @pl.when(cond_r)
                    def _(wr=wr):
                        wr.wait_recv()

                    cond_rm = p + h + 1 <= 3
                    wm = pltpu.make_async_remote_copy(
                        src_ref=sweep_slice(s, p + h + 1),
                        dst_ref=sweep_slice(s, p + h + 1),
                        send_sem=ms_sems.at[s, h],
                        recv_sem=mr_sems.at[s, h],
                        device_id=sweep_dev(s, -1),
                        device_id_type=pl.DeviceIdType.MESH,
                    )

                    @pl.when(cond_rm)
                    def _(wm=wm):
                        wm.wait_recv()

        run_phase([0, 1])
        run_phase([2, 3])

        for cond, r in deferred_sends:

            @pl.when(cond)
            def _(r=r):
                r.wait_send()

        for b in range(NY):
            buf = b % 2
            if b >= 2:
                pltpu.make_async_copy(
                    ostage_ref.at[buf],
                    out_ref.at[pl.ds((b - 2) * BLOCK, BLOCK), :],
                    osems.at[buf],
                ).wait()
            ostage_ref[buf] = jnp.reshape(ag_ref[b], (BLOCK, DIM)).astype(
                jnp.float32
            )
            pltpu.make_async_copy(
                ostage_ref.at[buf],
                out_ref.at[pl.ds(b * BLOCK, BLOCK), :],
                osems.at[buf],
            ).start()
        for b in (2, 3):
            buf = b % 2
            pltpu.make_async_copy(
                ostage_ref.at[buf],
                out_ref.at[pl.ds(b * BLOCK, BLOCK), :],
                osems.at[buf],
            ).wait()

    return pl.pallas_call(
        body,
        out_shape=jax.ShapeDtypeStruct((TOKENS, DIM), jnp.float32),
        in_specs=[
            pl.BlockSpec(memory_space=pltpu.SMEM),
            pl.BlockSpec(memory_space=pltpu.VMEM),
            pl.BlockSpec(memory_space=pl.ANY),
        ],
        out_specs=pl.BlockSpec(memory_space=pl.ANY),
        scratch_shapes=[
            pltpu.VMEM((NY, NZ, CHUNK, DIM), jnp.bfloat16),
            pltpu.VMEM((CHUNK, DIM), jnp.float32),
            pltpu.VMEM((1, 1, CHUNK, DIM), jnp.bfloat16),
            pltpu.VMEM((2, BLOCK, DIM), jnp.float32),
            pltpu.SemaphoreType.DMA,
            pltpu.SemaphoreType.DMA,
            pltpu.SemaphoreType.DMA,
            pltpu.SemaphoreType.DMA((N_SWEEP, N_HOP)),
            pltpu.SemaphoreType.DMA((N_SWEEP, N_HOP)),
            pltpu.SemaphoreType.DMA((N_SWEEP, N_HOP)),
            pltpu.SemaphoreType.DMA((N_SWEEP, N_HOP)),
            pltpu.SemaphoreType.DMA((2,)),
        ],
        compiler_params=pltpu.CompilerParams(
            vmem_limit_bytes=56 * 1024 * 1024,
        ),
    )(rows, mask, E)


# device time: 167349 ns/iter; 16.3254x vs baseline; 1.0031x over previous
import jax
import jax.numpy as jnp
from jax import lax
from jax.experimental import pallas as pl
from jax.experimental.pallas import tpu as pltpu

TOKENS = 4096
DIM = 2048
HALF = DIM // 2
NY = 4
NZ = 4
CHUNK = TOKENS // (NY * NZ)
BLOCK = NZ * CHUNK
N_SWEEP = 4
N_HOP = 3


def kernel(ids, E):
    vshard = E.shape[0]
    my_x = lax.axis_index("x")
    my_y = lax.axis_index("y")
    my_z = lax.axis_index("z")
    base = my_x * vshard

    rows_all = jnp.clip(ids - base, 0, vshard - 1).astype(jnp.int32)
    mask_all = ((ids >= base) & (ids < base + vshard)).astype(jnp.float32)
    c = my_y * NZ + my_z
    rows = lax.dynamic_slice(rows_all, (c * CHUNK,), (CHUNK,))
    mask = lax.dynamic_slice(mask_all, (c * CHUNK,), (CHUNK,))[:, None]

    def body(
        rows_ref,
        mask_ref,
        e_ref,
        out_ref,
        ag_ref,
        stage_ref,
        xrecv_ref,
        ostage_ref,
        gsem,
        xs_sem,
        xr_sem,
        ps_sems,
        pr_sems,
        ms_sems,
        mr_sems,
        osems,
    ):
        x = lax.axis_index("x")
        y = lax.axis_index("y")
        z = lax.axis_index("z")

        for t in range(CHUNK):
            r = rows_ref[t]
            pltpu.make_async_copy(
                e_ref.at[pl.ds(r, 1), :], stage_ref.at[pl.ds(t, 1), :], gsem
            ).start()
        for t in range(CHUNK):
            pltpu.make_async_copy(
                e_ref.at[pl.ds(0, 1), :], stage_ref.at[pl.ds(t, 1), :], gsem
            ).wait()

        home = ag_ref.at[pl.ds(y, 1), pl.ds(z, 1), :, :]
        ag_ref[pl.ds(y, 1), pl.ds(z, 1), :, :] = jnp.reshape(
            (stage_ref[...] * mask_ref[...]).astype(jnp.bfloat16),
            (1, 1, CHUNK, DIM),
        )

        x_rdma = pltpu.make_async_remote_copy(
            src_ref=home,
            dst_ref=xrecv_ref,
            send_sem=xs_sem,
            recv_sem=xr_sem,
            device_id=(1 - x, y, z),
            device_id_type=pl.DeviceIdType.MESH,
        )
        x_rdma.start()
        x_rdma.wait()
        ag_ref[pl.ds(y, 1), pl.ds(z, 1), :, :] = (
            ag_ref[pl.ds(y, 1), pl.ds(z, 1), :, :] + xrecv_ref[...]
        )

        def sweep_slice(s, b):
            bb = jnp.clip(b, 0, 3)
            if s == 0:
                return ag_ref.at[pl.ds(y, 1), pl.ds(bb, 1), :, pl.ds(0, HALF)]
            if s == 1:
                return ag_ref.at[pl.ds(bb, 1), pl.ds(z, 1), :, pl.ds(HALF, HALF)]
            if s == 2:
                return ag_ref.at[pl.ds(bb, 1), :, :, pl.ds(0, HALF)]
            return ag_ref.at[:, pl.ds(bb, 1), :, pl.ds(HALF, HALF)]

        def sweep_pos(s):
            return z if s in (0, 3) else y

        def sweep_dev(s, delta):
            if s in (0, 3):
                return (x, y, jnp.clip(z + delta, 0, NZ - 1))
            return (x, jnp.clip(y + delta, 0, NY - 1), z)

        deferred_sends = []

        def run_phase(sweeps):
            for h in range(N_HOP):
                for s in sweeps:
                    p = sweep_pos(s)
                    cond_p = (p < 3) & (p - h >= 0)
                    rp = pltpu.make_async_remote_copy(
                        src_ref=sweep_slice(s, p - h),
                        dst_ref=sweep_slice(s, p - h),
                        send_sem=ps_sems.at[s, h],
                        recv_sem=pr_sems.at[s, h],
                        device_id=sweep_dev(s, 1),
                        device_id_type=pl.DeviceIdType.MESH,
                    )

                    @pl.when(cond_p)
                    def _(rp=rp):
                        rp.start()

                    deferred_sends.append((cond_p, rp))

                    cond_m = (p > 0) & (p + h <= 3)
                    rm = pltpu.make_async_remote_copy(
                        src_ref=sweep_slice(s, p + h),
                        dst_ref=sweep_slice(s, p + h),
                        send_sem=ms_sems.at[s, h],
                        recv_sem=mr_sems.at[s, h],
                        device_id=sweep_dev(s, -1),
                        device_id_type=pl.DeviceIdType.MESH,
                    )

                    @pl.when(cond_m)
                    def _(rm=rm):
                        rm.start()

                    deferred_sends.append((cond_m, rm))

                for s in sweeps:
                    p = sweep_pos(s)
                    cond_r = p - h - 1 >= 0
                    wr = pltpu.make_async_remote_copy(
                        src_ref=sweep_slice(s, p - h - 1),
                        dst_ref=sweep_slice(s, p - h - 1),
                        send_sem=ps_sems.at[s, h],
                        recv_sem=pr_sems.at[s, h],
                        device_id=sweep_dev(s, 1),
                        device_id_type=pl.DeviceIdType.MESH,
                    )

                    @pl.when(cond_r)
                    def _(wr=wr):
                        wr.wait_recv()

                    cond_rm = p + h + 1 <= 3
                    wm = pltpu.make_async_remote_copy(
                        src_ref=sweep_slice(s, p + h + 1),
                        dst_ref=sweep_slice(s, p + h + 1),
                        send_sem=ms_sems.at[s, h],
                        recv_sem=mr_sems.at[s, h],
                        device_id=sweep_dev(s, -1),
                        device_id_type=pl.DeviceIdType.MESH,
                    )

                    @pl.when(cond_rm)
                    def _(wm=wm):
                        wm.wait_recv()

        run_phase([0, 1])
        run_phase([2, 3])

        for cond, r in deferred_sends:

            @pl.when(cond)
            def _(r=r):
                r.wait_send()

        for b in range(NY):
            buf = b % 2
            if b >= 2:
                pltpu.make_async_copy(
                    ostage_ref.at[buf],
                    out_ref.at[pl.ds((b - 2) * BLOCK, BLOCK), :],
                    osems.at[buf],
                ).wait()
            ostage_ref[buf] = jnp.reshape(ag_ref[b], (BLOCK, DIM)).astype(
                jnp.float32
            )
            pltpu.make_async_copy(
                ostage_ref.at[buf],
                out_ref.at[pl.ds(b * BLOCK, BLOCK), :],
                osems.at[buf],
            ).start()
        for b in (2, 3):
            buf = b % 2
            pltpu.make_async_copy(
                ostage_ref.at[buf],
                out_ref.at[pl.ds(b * BLOCK, BLOCK), :],
                osems.at[buf],
            ).wait()

    return pl.pallas_call(
        body,
        out_shape=jax.ShapeDtypeStruct((TOKENS, DIM), jnp.float32),
        in_specs=[
            pl.BlockSpec(memory_space=pltpu.SMEM),
            pl.BlockSpec(memory_space=pltpu.VMEM),
            pl.BlockSpec(memory_space=pl.ANY),
        ],
        out_specs=pl.BlockSpec(memory_space=pl.ANY),
        scratch_shapes=[
            pltpu.VMEM((NY, NZ, CHUNK, DIM), jnp.bfloat16),
            pltpu.VMEM((CHUNK, DIM), jnp.float32),
            pltpu.VMEM((1, 1, CHUNK, DIM), jnp.bfloat16),
            pltpu.VMEM((2, BLOCK, DIM), jnp.float32),
            pltpu.SemaphoreType.DMA,
            pltpu.SemaphoreType.DMA,
            pltpu.SemaphoreType.DMA,
            pltpu.SemaphoreType.DMA((N_SWEEP, N_HOP)),
            pltpu.SemaphoreType.DMA((N_SWEEP, N_HOP)),
            pltpu.SemaphoreType.DMA((N_SWEEP, N_HOP)),
            pltpu.SemaphoreType.DMA((N_SWEEP, N_HOP)),
            pltpu.SemaphoreType.DMA((2,)),
        ],
        compiler_params=pltpu.CompilerParams(
            vmem_limit_bytes=56 * 1024 * 1024,
        ),
    )(rows, mask, E)


# device time: 140835 ns/iter; 19.3989x vs baseline; 1.1883x over previous
import os

import jax
import jax.numpy as jnp
from jax import lax
from jax.experimental import pallas as pl
from jax.experimental.pallas import tpu as pltpu

TOKENS = 4096
DIM = 2048
HALF = DIM // 2
NY = 4
NZ = 4
CHUNK = TOKENS // (NY * NZ)
BLOCK = NZ * CHUNK
N_SWEEP = 4
N_HOP = 3


def kernel(ids, E):
    vshard = E.shape[0]
    my_x = lax.axis_index("x")
    my_y = lax.axis_index("y")
    my_z = lax.axis_index("z")
    base = my_x * vshard

    rows_all = jnp.clip(ids - base, 0, vshard - 1).astype(jnp.int32)
    mask_all = ((ids >= base) & (ids < base + vshard)).astype(jnp.int32)
    c = my_y * NZ + my_z
    rows = lax.dynamic_slice(rows_all, (c * CHUNK,), (CHUNK,))
    lmask = lax.dynamic_slice(mask_all, (c * CHUNK,), (CHUNK,))
    nloc = jnp.sum(lmask).astype(jnp.int32)[None]

    def body(
        rows_ref,
        lmask_ref,
        nloc_ref,
        e_ref,
        out_ref,
        ag_ref,
        stage_ref,
        xrecv_ref,
        o1_ref,
        o2_ref,
        gsem,
        xs_sem,
        xr_sem,
        ps_sems,
        pr_sems,
        ms_sems,
        mr_sems,
        xfs_sems,
        xfr_sems,
        o1_sems,
        o2_sems,
    ):
        x = lax.axis_index("x")
        y = lax.axis_index("y")
        z = lax.axis_index("z")

        probe = int(os.environ.get("KERNEL_PROBE", "0"))

        if probe != 6:
            barrier_sem = pltpu.get_barrier_semaphore()
            pl.semaphore_signal(
                barrier_sem,
                inc=1,
                device_id=(1 - x, y, z),
                device_id_type=pl.DeviceIdType.MESH,
            )
        if probe != 6:
            for dy in (-1, 1):
                @pl.when((y + dy >= 0) & (y + dy <= NY - 1))
                def _(dy=dy):
                    pl.semaphore_signal(
                        barrier_sem,
                        inc=1,
                        device_id=(x, jnp.clip(y + dy, 0, NY - 1), z),
                        device_id_type=pl.DeviceIdType.MESH,
                    )
            for dz in (-1, 1):
                @pl.when((z + dz >= 0) & (z + dz <= NZ - 1))
                def _(dz=dz):
                    pl.semaphore_signal(
                        barrier_sem,
                        inc=1,
                        device_id=(x, y, jnp.clip(z + dz, 0, NZ - 1)),
                        device_id_type=pl.DeviceIdType.MESH,
                    )
            n_nbr_t = (
                1
                + (y > 0).astype(jnp.int32)
                + (y < NY - 1).astype(jnp.int32)
                + (z > 0).astype(jnp.int32)
                + (z < NZ - 1).astype(jnp.int32)
            )
            pl.semaphore_wait(barrier_sem, n_nbr_t)

        stage_ref[...] = jnp.zeros((CHUNK, DIM), jnp.float32)
        if probe not in (4, 5, 6):
            for t in range(CHUNK):
                @pl.when(lmask_ref[t] == 1)
                def _(t=t):
                    pltpu.make_async_copy(
                        e_ref.at[pl.ds(rows_ref[t], 1), :],
                        stage_ref.at[pl.ds(t, 1), :],
                        gsem,
                    ).start()

            def drain(i, _):
                pltpu.make_async_copy(
                    e_ref.at[pl.ds(0, 1), :], stage_ref.at[pl.ds(0, 1), :], gsem
                ).wait()
                return 0

            lax.fori_loop(0, nloc_ref[0], drain, 0)

        ag_ref[pl.ds(y, 1), pl.ds(z, 1), :, :] = jnp.reshape(
            stage_ref[...].astype(jnp.bfloat16), (1, 1, CHUNK, DIM)
        )

        deferred_sends = []
        x_rdmas = []
        if probe not in (3, 5, 6):
            for i in range(2):
                cols = pl.ds(i * HALF, HALF)
                x_rdmas.append(
                    pltpu.make_async_remote_copy(
                        src_ref=ag_ref.at[pl.ds(y, 1), pl.ds(z, 1), :, cols],
                        dst_ref=xrecv_ref.at[:, :, :, cols],
                        send_sem=xs_sem.at[i],
                        recv_sem=xr_sem.at[i],
                        device_id=(1 - x, y, z),
                        device_id_type=pl.DeviceIdType.MESH,
                    )
                )
            for r in x_rdmas:
                r.start()
                deferred_sends.append((y == y, r))

        def x_sum(i):
            cols = pl.ds(i * HALF, HALF)
            x_rdmas[i].wait_recv()
            ag_ref[pl.ds(y, 1), pl.ds(z, 1), :, cols] = (
                ag_ref[pl.ds(y, 1), pl.ds(z, 1), :, cols]
                + xrecv_ref[:, :, :, cols]
            )

        PW = 768
        FW = HALF - PW

        def colrange(s, kind):
            hbase = 0 if s in (0, 2) else HALF
            if kind == "plane":
                return pl.ds(hbase + FW * x, PW)
            return pl.ds(hbase + PW * x, FW)

        def sweep_slice(s, b, kind="plane"):
            bb = jnp.clip(b, 0, 3)
            cols = colrange(s, kind)
            if s == 0:
                return ag_ref.at[pl.ds(y, 1), pl.ds(bb, 1), :, cols]
            if s == 1:
                return ag_ref.at[pl.ds(bb, 1), pl.ds(z, 1), :, cols]
            if s == 2:
                return ag_ref.at[pl.ds(bb, 1), :, :, cols]
            return ag_ref.at[:, pl.ds(bb, 1), :, cols]

        def sweep_pos(s):
            return z if s in (0, 3) else y

        def sweep_dev(s, delta):
            if s in (0, 3):
                return (x, y, jnp.clip(z + delta, 0, NZ - 1))
            return (x, jnp.clip(y + delta, 0, NY - 1), z)

        def hop_starts(sweeps, h):
            for s in sweeps:
                p = sweep_pos(s)
                cond_p = (p < 3) & (p - h >= 0)
                rp = pltpu.make_async_remote_copy(
                    src_ref=sweep_slice(s, p - h),
                    dst_ref=sweep_slice(s, p - h),
                    send_sem=ps_sems.at[s, h],
                    recv_sem=pr_sems.at[s, h],
                    device_id=sweep_dev(s, 1),
                    device_id_type=pl.DeviceIdType.MESH,
                )

                @pl.when(cond_p)
                def _(rp=rp):
                    rp.start()

                deferred_sends.append((cond_p, rp))

                cond_m = (p > 0) & (p + h <= 3)
                rm = pltpu.make_async_remote_copy(
                    src_ref=sweep_slice(s, p + h),
                    dst_ref=sweep_slice(s, p + h),
                    send_sem=ms_sems.at[s, h],
                    recv_sem=mr_sems.at[s, h],
                    device_id=sweep_dev(s, -1),
                    device_id_type=pl.DeviceIdType.MESH,
                )

                @pl.when(cond_m)
                def _(rm=rm):
                    rm.start()

                deferred_sends.append((cond_m, rm))

        def xfwd_rdma(s, h, d, b):
            return pltpu.make_async_remote_copy(
                src_ref=sweep_slice(s, b, "fwd"),
                dst_ref=sweep_slice(s, b, "fwd"),
                send_sem=xfs_sems.at[s, h, d],
                recv_sem=xfr_sems.at[s, h, d],
                device_id=(1 - x, y, z),
                device_id_type=pl.DeviceIdType.MESH,
            )

        def wait_xfr(s, h, d, cond):
            w = xfwd_rdma(s, h, d, 0)

            @pl.when(cond)
            def _(w=w):
                w.wait_recv()

        def hop_waits(sweeps, h):
            for s in sweeps:
                p = sweep_pos(s)
                cond_r = p - h - 1 >= 0
                wr = pltpu.make_async_remote_copy(
                    src_ref=sweep_slice(s, p - h - 1),
                    dst_ref=sweep_slice(s, p - h - 1),
                    send_sem=ps_sems.at[s, h],
                    recv_sem=pr_sems.at[s, h],
                    device_id=sweep_dev(s, 1),
                    device_id_type=pl.DeviceIdType.MESH,
                )

                @pl.when(cond_r)
                def _(wr=wr):
                    wr.wait_recv()

                fp = xfwd_rdma(s, h, 0, p - h - 1)

                @pl.when(cond_r)
                def _(fp=fp):
                    fp.start()

                deferred_sends.append((cond_r, fp))

                cond_rm = p + h + 1 <= 3
                wm = pltpu.make_async_remote_copy(
                    src_ref=sweep_slice(s, p + h + 1),
                    dst_ref=sweep_slice(s, p + h + 1),
                    send_sem=ms_sems.at[s, h],
                    recv_sem=mr_sems.at[s, h],
                    device_id=sweep_dev(s, -1),
                    device_id_type=pl.DeviceIdType.MESH,
                )

                @pl.when(cond_rm)
                def _(wm=wm):
                    wm.wait_recv()

                fm = xfwd_rdma(s, h, 1, p + h + 1)

                @pl.when(cond_rm)
                def _(fm=fm):
                    fm.start()

                deferred_sends.append((cond_rm, fm))

        o1_prev = [None, None]
        o2_prev = [None, None]

        def o1_descs(buf, yb):
            return [
                pltpu.make_async_copy(
                    o1_ref.at[buf],
                    out_ref.at[pl.ds(yb * BLOCK, BLOCK), pl.ds(0, HALF)],
                    o1_sems.at[buf],
                )
            ]

        def o2_descs(buf, w):
            return [
                pltpu.make_async_copy(
                    o2_ref.at[buf, yy],
                    out_ref.at[
                        pl.ds((yy * NZ + w) * CHUNK, CHUNK), pl.ds(HALF, HALF)
                    ],
                    o2_sems.at[buf],
                )
                for yy in range(NY)
            ]

        def store_half1(b, cond, buf):
            yb = jnp.clip(b, 0, 3)
            prev = o1_prev[buf]
            if prev is not None:
                prev_cond, prev_yb = prev

                @pl.when(prev_cond)
                def _():
                    for d in o1_descs(buf, prev_yb):
                        d.wait()

            @pl.when(cond)
            def _():
                o1_ref[buf] = jnp.reshape(
                    ag_ref[pl.ds(yb, 1), :, :, pl.ds(0, HALF)], (BLOCK, HALF)
                ).astype(jnp.float32)
                for d in o1_descs(buf, yb):
                    d.start()

            o1_prev[buf] = (cond, yb)

        def store_half2(b, cond, buf):
            w = jnp.clip(b, 0, 3)
            prev = o2_prev[buf]
            if prev is not None:
                prev_cond, prev_w = prev

                @pl.when(prev_cond)
                def _():
                    for d in o2_descs(buf, prev_w):
                        d.wait()

            @pl.when(cond)
            def _():
                o2_ref[buf] = jnp.reshape(
                    ag_ref[:, pl.ds(w, 1), :, pl.ds(HALF, HALF)],
                    (NY, CHUNK, HALF),
                ).astype(jnp.float32)
                for d in o2_descs(buf, w):
                    d.start()

            o2_prev[buf] = (cond, w)

        if probe == 1 or probe == 0:
            x_sum(0)
            hop_starts([0], 0)
            x_sum(1)
            hop_starts([1], 0)
            hop_waits([0, 1], 0)
            for h in range(1, N_HOP):
                hop_starts([0, 1], h)
                hop_waits([0, 1], h)

        if probe == 0:
            for h in range(N_HOP):
                hop_starts([2, 3], h)
                if h == 0:
                    for hh in range(N_HOP):
                        wait_xfr(0, hh, 0, z - hh - 1 >= 0)
                        wait_xfr(0, hh, 1, z + hh + 1 <= 3)
                        wait_xfr(1, hh, 0, y - hh - 1 >= 0)
                        wait_xfr(1, hh, 1, y + hh + 1 <= 3)
                    store_half1(y, y == y, 0)
                    store_half2(z, z == z, 0)
                else:
                    wait_xfr(2, h - 1, 0, y - h >= 0)
                    wait_xfr(2, h - 1, 1, y + h <= 3)
                    wait_xfr(3, h - 1, 0, z - h >= 0)
                    wait_xfr(3, h - 1, 1, z + h <= 3)
                    store_half1(y - h, y - h >= 0, 0)
                    store_half1(y + h, y + h <= 3, 1)
                    store_half2(z - h, z - h >= 0, 0)
                    store_half2(z + h, z + h <= 3, 1)
                hop_waits([2, 3], h)
            wait_xfr(2, N_HOP - 1, 0, y - N_HOP >= 0)
            wait_xfr(2, N_HOP - 1, 1, y + N_HOP <= 3)
            wait_xfr(3, N_HOP - 1, 0, z - N_HOP >= 0)
            wait_xfr(3, N_HOP - 1, 1, z + N_HOP <= 3)
            store_half1(y - N_HOP, y - N_HOP >= 0, 0)
            store_half1(y + N_HOP, y + N_HOP <= 3, 1)
            store_half2(z - N_HOP, z - N_HOP >= 0, 0)
            store_half2(z + N_HOP, z + N_HOP <= 3, 1)

            for buf in range(2):
                prev_cond, prev_yb = o1_prev[buf]

                @pl.when(prev_cond)
                def _(buf=buf, prev_yb=prev_yb):
                    for d in o1_descs(buf, prev_yb):
                        d.wait()

                prev_cond2, prev_w = o2_prev[buf]

                @pl.when(prev_cond2)
                def _(buf=buf, prev_w=prev_w):
                    for d in o2_descs(buf, prev_w):
                        d.wait()

        for cond, r in deferred_sends:

            @pl.when(cond)
            def _(r=r):
                r.wait_send()

    return pl.pallas_call(
        body,
        out_shape=jax.ShapeDtypeStruct((TOKENS, DIM), jnp.float32),
        in_specs=[
            pl.BlockSpec(memory_space=pltpu.SMEM),
            pl.BlockSpec(memory_space=pltpu.SMEM),
            pl.BlockSpec(memory_space=pltpu.SMEM),
            pl.BlockSpec(memory_space=pl.ANY),
        ],
        out_specs=pl.BlockSpec(memory_space=pl.ANY),
        scratch_shapes=[
            pltpu.VMEM((NY, NZ, CHUNK, DIM), jnp.bfloat16),
            pltpu.VMEM((CHUNK, DIM), jnp.float32),
            pltpu.VMEM((1, 1, CHUNK, DIM), jnp.bfloat16),
            pltpu.VMEM((2, BLOCK, HALF), jnp.float32),
            pltpu.VMEM((2, NY, CHUNK, HALF), jnp.float32),
            pltpu.SemaphoreType.DMA,
            pltpu.SemaphoreType.DMA((2,)),
            pltpu.SemaphoreType.DMA((2,)),
            pltpu.SemaphoreType.DMA((N_SWEEP, N_HOP)),
            pltpu.SemaphoreType.DMA((N_SWEEP, N_HOP)),
            pltpu.SemaphoreType.DMA((N_SWEEP, N_HOP)),
            pltpu.SemaphoreType.DMA((N_SWEEP, N_HOP)),
            pltpu.SemaphoreType.DMA((N_SWEEP, N_HOP, 2)),
            pltpu.SemaphoreType.DMA((N_SWEEP, N_HOP, 2)),
            pltpu.SemaphoreType.DMA((2,)),
            pltpu.SemaphoreType.DMA((2,)),
        ],
        compiler_params=pltpu.CompilerParams(
            vmem_limit_bytes=56 * 1024 * 1024,
            collective_id=0,
        ),
    )(rows, lmask, nloc, E)
